# baseline (speedup 1.0000x reference)
import functools

import jax
import jax.numpy as jnp
from jax import lax
from jax.experimental import pallas as pl
from jax.experimental.pallas import tpu as pltpu
from jax.experimental.pallas import tpu_sc as plsc

V = 1000
D = 32
DP = 128
T = 8
B = 4096

NC, NS = 2, 16
NW = NC * NS
BTOT = B * T
BPW = BTOT // NW
ROWS = 128
NBUF = 4
NCH = BPW // ROWS
NGRP = NCH // NBUF

BB = 2048
NJ = B // BB


def _sc_gather_body(emb_hbm, idx_hbm, out_hbm, idx_v, bufs_v, gsem, ssem):
    wid = lax.axis_index("s") * NC + lax.axis_index("c")
    base = wid * BPW

    pltpu.sync_copy(idx_hbm.at[pl.ds(base, BPW)], idx_v)

    def issue_gather(chunk, b):
        pltpu.async_copy(
            emb_hbm.at[idx_v.at[pl.ds(chunk * ROWS, ROWS)]], bufs_v.at[b], gsem
        )

    def wait_gather(chunk, b):
        pltpu.make_async_copy(
            emb_hbm.at[idx_v.at[pl.ds(chunk * ROWS, ROWS)]], bufs_v.at[b], gsem
        ).wait()

    def issue_scatter(chunk, b):
        pltpu.async_copy(
            bufs_v.at[b], out_hbm.at[pl.ds(base + chunk * ROWS, ROWS)], ssem
        )

    def wait_scatter(chunk, b):
        pltpu.make_async_copy(
            bufs_v.at[b], out_hbm.at[pl.ds(base + chunk * ROWS, ROWS)], ssem
        ).wait()

    for b in range(NBUF):
        issue_gather(b, b)
    for j in range(NGRP):
        g0 = j * NBUF
        for b in range(NBUF):
            wait_gather(g0 + b, b)
        for b in range(NBUF):
            issue_scatter(g0 + b, b)
        for b in range(NBUF):
            wait_scatter(g0 + b, b)
        if j + 1 < NGRP:
            for b in range(NBUF):
                issue_gather(g0 + NBUF + b, b)


@functools.cache
def _sc_gather():
    mesh = plsc.VectorSubcoreMesh(
        core_axis_name="c", subcore_axis_name="s", num_cores=NC, num_subcores=NS
    )
    return pl.kernel(
        _sc_gather_body,
        out_type=jax.ShapeDtypeStruct((BTOT, DP), jnp.float32),
        mesh=mesh,
        scratch_types=[
            pltpu.VMEM((BPW,), jnp.int32),
            pltpu.VMEM((NBUF, ROWS, DP), jnp.float32),
            pltpu.SemaphoreType.DMA,
            pltpu.SemaphoreType.DMA,
        ],
    )


def _head_body(x_ref, w_ref, pos_ref, bias_ref, out_ref):
    xb = x_ref[:].reshape(BB, DP).astype(jnp.bfloat16)
    y = lax.dot_general(
        w_ref[:], xb, (((0,), (1,)), ((), ())),
        preferred_element_type=jnp.float32,
    )
    posw = lax.dot_general(
        w_ref[:], pos_ref[:].reshape(1, DP), (((0,), (1,)), ((), ())),
        preferred_element_type=jnp.float32,
    )
    out_ref[:] = (y + (posw + bias_ref[:])).reshape(1, V, BB)


def _head(x3, w_pad_bf16, pos_pad_bf16, bias_col):
    return pl.pallas_call(
        _head_body,
        grid=(T, NJ),
        in_specs=[
            pl.BlockSpec((1, BB, DP), lambda t, j: (t, j, 0)),
            pl.BlockSpec((DP, V), lambda t, j: (0, 0)),
            pl.BlockSpec((1, 1, DP), lambda t, j: (t, 0, 0)),
            pl.BlockSpec((V, 1), lambda t, j: (0, 0)),
        ],
        out_specs=pl.BlockSpec((1, V, BB), lambda t, j: (t, 0, j)),
        out_shape=jax.ShapeDtypeStruct((T, V, B), jnp.float32),
    )(x3, w_pad_bf16, pos_pad_bf16, bias_col)


def kernel(idx, embedding, positional_embedding, lm_head_w, lm_head_b):
    emb_pad = jnp.pad(embedding, ((0, 0), (0, DP - D)))
    pos_pad = jnp.pad(positional_embedding, ((0, 0), (0, DP - D))).astype(jnp.bfloat16)
    w_pad = jnp.pad(lm_head_w, ((0, DP - D), (0, 0))).astype(jnp.bfloat16)
    idx_tmajor = idx.T.reshape(BTOT).astype(jnp.int32)
    x = _sc_gather()(emb_pad, idx_tmajor)
    out_t = _head(x.reshape(T, B, DP), w_pad, pos_pad.reshape(T, 1, DP),
                  lm_head_b.reshape(V, 1))
    return jnp.transpose(out_t, (2, 0, 1))


# speedup vs baseline: 1.0094x; 1.0094x over previous
import functools

import jax
import jax.numpy as jnp
from jax import lax
from jax.experimental import pallas as pl
from jax.experimental.pallas import tpu as pltpu
from jax.experimental.pallas import tpu_sc as plsc

V = 1000
D = 32
DP = 128
T = 8
B = 4096

NC, NS = 2, 16
NW = NC * NS
BTOT = B * T
BPW = BTOT // NW
ROWS = 128
NBUF = 4
NCH = BPW // ROWS
NGRP = NCH // NBUF

BB = 4096
NJ = B // BB


def _sc_gather_body(emb_hbm, idx_hbm, out_hbm, idx_v, bufs_v, gsem, ssem):
    wid = lax.axis_index("s") * NC + lax.axis_index("c")
    base = wid * BPW

    pltpu.sync_copy(idx_hbm.at[pl.ds(base, BPW)], idx_v)

    def issue_gather(chunk, b):
        pltpu.async_copy(
            emb_hbm.at[idx_v.at[pl.ds(chunk * ROWS, ROWS)]], bufs_v.at[b], gsem
        )

    def wait_gather(chunk, b):
        pltpu.make_async_copy(
            emb_hbm.at[idx_v.at[pl.ds(chunk * ROWS, ROWS)]], bufs_v.at[b], gsem
        ).wait()

    def issue_scatter(chunk, b):
        pltpu.async_copy(
            bufs_v.at[b], out_hbm.at[pl.ds(base + chunk * ROWS, ROWS)], ssem
        )

    def wait_scatter(chunk, b):
        pltpu.make_async_copy(
            bufs_v.at[b], out_hbm.at[pl.ds(base + chunk * ROWS, ROWS)], ssem
        ).wait()

    for b in range(NBUF):
        issue_gather(b, b)
    for j in range(NGRP):
        g0 = j * NBUF
        for b in range(NBUF):
            wait_gather(g0 + b, b)
        for b in range(NBUF):
            issue_scatter(g0 + b, b)
        for b in range(NBUF):
            wait_scatter(g0 + b, b)
        if j + 1 < NGRP:
            for b in range(NBUF):
                issue_gather(g0 + NBUF + b, b)


@functools.cache
def _sc_gather():
    mesh = plsc.VectorSubcoreMesh(
        core_axis_name="c", subcore_axis_name="s", num_cores=NC, num_subcores=NS
    )
    return pl.kernel(
        _sc_gather_body,
        out_type=jax.ShapeDtypeStruct((BTOT, DP), jnp.float32),
        mesh=mesh,
        scratch_types=[
            pltpu.VMEM((BPW,), jnp.int32),
            pltpu.VMEM((NBUF, ROWS, DP), jnp.float32),
            pltpu.SemaphoreType.DMA,
            pltpu.SemaphoreType.DMA,
        ],
    )


def _head_body(x_ref, w_ref, pos_ref, bias_ref, out_ref):
    xb = x_ref[:].reshape(BB, DP).astype(jnp.bfloat16)
    y = lax.dot_general(
        w_ref[:], xb, (((0,), (1,)), ((), ())),
        preferred_element_type=jnp.float32,
    )
    posw = lax.dot_general(
        w_ref[:], pos_ref[:].reshape(1, DP), (((0,), (1,)), ((), ())),
        preferred_element_type=jnp.float32,
    )
    out_ref[:] = (y + (posw + bias_ref[:])).reshape(1, V, BB)


def _head(x3, w_pad_bf16, pos_pad_bf16, bias_col):
    return pl.pallas_call(
        _head_body,
        grid=(T, NJ),
        in_specs=[
            pl.BlockSpec((1, BB, DP), lambda t, j: (t, j, 0)),
            pl.BlockSpec((DP, V), lambda t, j: (0, 0)),
            pl.BlockSpec((1, 1, DP), lambda t, j: (t, 0, 0)),
            pl.BlockSpec((V, 1), lambda t, j: (0, 0)),
        ],
        out_specs=pl.BlockSpec((1, V, BB), lambda t, j: (t, 0, j)),
        out_shape=jax.ShapeDtypeStruct((T, V, B), jnp.float32),
    )(x3, w_pad_bf16, pos_pad_bf16, bias_col)


def kernel(idx, embedding, positional_embedding, lm_head_w, lm_head_b):
    emb_pad = jnp.pad(embedding, ((0, 0), (0, DP - D)))
    pos_pad = jnp.pad(positional_embedding, ((0, 0), (0, DP - D))).astype(jnp.bfloat16)
    w_pad = jnp.pad(lm_head_w, ((0, DP - D), (0, 0))).astype(jnp.bfloat16)
    idx_tmajor = idx.T.reshape(BTOT).astype(jnp.int32)
    x = _sc_gather()(emb_pad, idx_tmajor)
    out_t = _head(x.reshape(T, B, DP), w_pad, pos_pad.reshape(T, 1, DP),
                  lm_head_b.reshape(V, 1))
    return jnp.transpose(out_t, (2, 0, 1))
